# baseline (speedup 1.0000x reference)
import functools

import jax
import jax.numpy as jnp
from jax import lax
from jax.experimental import pallas as pl
from jax.experimental.pallas import tpu as pltpu
from jax.experimental.pallas import tpu_sc as plsc

N_NODES = 10000
N_EDGES = 320000
D = 128
N_GRAPHS = 64
N_CLASSES = 10

NC = 2
NS = 16
NW = NC * NS

CHUNK = 128
CH_PER_TILE = 80
IDX_HALF = CH_PER_TILE // 2
EDGES_PER_TILE = CH_PER_TILE * CHUNK
PAD_EDGES = NW * EDGES_PER_TILE
TRASH_ROW = N_NODES + 8
ACC_ROWS = 10240
ROWS_PER_TILE = ACC_ROWS // NS
WB_ROWS = 624

DEG_W = 16

_mesh = plsc.VectorSubcoreMesh(core_axis_name="c", subcore_axis_name="s")


def _fill_vmem(ref, rows, width, value):
    vec = jnp.full((16,), value, jnp.float32)

    def body(i, _):
        for j in range(width // 16):
            ref[i, pl.ds(j * 16, 16)] = vec
        return 0

    lax.fori_loop(0, rows, body, 0)


def _zero_spmem_slice(tile_id, zbuf, table):
    n_full = ROWS_PER_TILE // CHUNK
    base = tile_id * ROWS_PER_TILE
    for k in range(n_full):
        pltpu.sync_copy(zbuf, table.at[pl.ds(base + k * CHUNK, CHUNK)])


def _writeback(c, s, table, out_hbm):
    base = s * WB_ROWS
    pltpu.sync_copy(table.at[pl.ds(base, WB_ROWS)],
                    out_hbm.at[c, pl.ds(base, WB_ROWS)])

    @pl.when(s == NS - 1)
    def _():
        tail = NS * WB_ROWS
        pltpu.sync_copy(table.at[pl.ds(tail, N_NODES - tail)],
                        out_hbm.at[c, pl.ds(tail, N_NODES - tail)])


@functools.partial(
    pl.kernel,
    out_type=jax.ShapeDtypeStruct((NC, N_NODES, DEG_W), jnp.float32),
    mesh=_mesh,
    scratch_types=[
        pltpu.VMEM((CH_PER_TILE, CHUNK), jnp.int32),
        pltpu.VMEM((CHUNK, DEG_W), jnp.float32),
        pltpu.VMEM_SHARED((ACC_ROWS, DEG_W), jnp.float32),
    ],
)
def _deg_kernel(dst_hbm, out_hbm, dstv, ones_v, table):
    c = lax.axis_index("c")
    s = lax.axis_index("s")
    wid = c * NS + s

    pltpu.sync_copy(dst_hbm.at[wid], dstv)

    _fill_vmem(ones_v, CHUNK, DEG_W, 0.0)
    _zero_spmem_slice(s, ones_v, table)
    _fill_vmem(ones_v, CHUNK, DEG_W, 1.0)

    plsc.subcore_barrier()

    def body(j, _):
        pltpu.sync_copy(ones_v, table.at[dstv.at[j]], add=True)
        return 0

    lax.fori_loop(0, CH_PER_TILE, body, 0)

    plsc.subcore_barrier()

    _writeback(c, s, table, out_hbm)


@functools.partial(
    pl.kernel,
    out_type=jax.ShapeDtypeStruct((NC, N_NODES, D), jnp.float32),
    mesh=_mesh,
    scratch_types=[
        pltpu.VMEM((CH_PER_TILE, CHUNK), jnp.int32),
        pltpu.VMEM((CH_PER_TILE, CHUNK), jnp.int32),
        pltpu.VMEM((CHUNK, D), jnp.float32),
        pltpu.VMEM_SHARED((ACC_ROWS, D), jnp.float32),
        pltpu.SemaphoreType.DMA,
    ],
)
def _agg_kernel(g_hbm, src_hbm, dst_hbm, out_hbm, srcv, dstv, bufa,
                acc, sema):
    c = lax.axis_index("c")
    s = lax.axis_index("s")
    wid = c * NS + s

    pltpu.sync_copy(src_hbm.at[wid], srcv)
    pltpu.sync_copy(dst_hbm.at[wid], dstv)

    _fill_vmem(bufa, CHUNK, D, 0.0)
    _zero_spmem_slice(s, bufa, acc)

    plsc.subcore_barrier()

    def body(j, _):
        pltpu.async_copy(g_hbm.at[srcv.at[j]], bufa, sema).wait()
        pltpu.sync_copy(bufa, acc.at[dstv.at[j]], add=True)
        return 0

    lax.fori_loop(0, CH_PER_TILE, body, 0)

    plsc.subcore_barrier()

    _writeback(c, s, acc, out_hbm)


ROW_BLK = 1000


def _l1_body(degp_ref, x_ref, w_ref, g_ref, dis_ref):
    deg = degp_ref[0, :, 0:1] + degp_ref[1, :, 0:1] + 1.0
    dis = lax.rsqrt(deg)
    h = jnp.dot(x_ref[...], w_ref[...], preferred_element_type=jnp.float32)
    g_ref[...] = dis * h
    dis_ref[...] = dis


def _layer1(degp, x, W1):
    return pl.pallas_call(
        _l1_body,
        grid=(N_NODES // ROW_BLK,),
        in_specs=[
            pl.BlockSpec((NC, ROW_BLK, DEG_W), lambda i: (0, i, 0)),
            pl.BlockSpec((ROW_BLK, D), lambda i: (i, 0)),
            pl.BlockSpec((D, D), lambda i: (0, 0)),
        ],
        out_specs=[
            pl.BlockSpec((ROW_BLK, D), lambda i: (i, 0)),
            pl.BlockSpec((ROW_BLK, 1), lambda i: (i, 0)),
        ],
        out_shape=[
            jax.ShapeDtypeStruct((N_NODES, D), jnp.float32),
            jax.ShapeDtypeStruct((N_NODES, 1), jnp.float32),
        ],
    )(degp, x, W1)


def _mid_body(accp_ref, g_ref, dis_ref, b_ref, w_ref, out_ref):
    acc = accp_ref[0] + accp_ref[1]
    dis = dis_ref[...]
    a = jnp.maximum(dis * (acc + g_ref[...]) + b_ref[...], 0.0)
    out_ref[...] = dis * jnp.dot(a, w_ref[...],
                                 preferred_element_type=jnp.float32)


def _mid_layer(accp, g, dis, b, W):
    return pl.pallas_call(
        _mid_body,
        grid=(N_NODES // ROW_BLK,),
        in_specs=[
            pl.BlockSpec((NC, ROW_BLK, D), lambda i: (0, i, 0)),
            pl.BlockSpec((ROW_BLK, D), lambda i: (i, 0)),
            pl.BlockSpec((ROW_BLK, 1), lambda i: (i, 0)),
            pl.BlockSpec((1, D), lambda i: (0, 0)),
            pl.BlockSpec((D, D), lambda i: (0, 0)),
        ],
        out_specs=pl.BlockSpec((ROW_BLK, D), lambda i: (i, 0)),
        out_shape=jax.ShapeDtypeStruct((N_NODES, D), jnp.float32),
    )(accp, g, dis, b.reshape(1, D), W)


def _final_body(accp_ref, g_ref, dis_ref, b_ref, batch_ref, wlin_ref,
                blin_ref, out_ref):
    acc = accp_ref[0] + accp_ref[1]
    dis = dis_ref[...]
    a = jnp.maximum(dis * (acc + g_ref[...]) + b_ref[...], 0.0)
    batch_row = batch_ref[0:1, :]
    gids = lax.broadcasted_iota(jnp.int32, (N_GRAPHS, N_NODES), 0)
    mask = (gids == batch_row).astype(jnp.float32)
    sums = jnp.dot(mask, a, preferred_element_type=jnp.float32)
    cnts = jnp.sum(mask, axis=1, keepdims=True)
    pooled = sums / jnp.maximum(cnts, 1.0)
    out_ref[...] = jnp.dot(pooled, wlin_ref[...],
                           preferred_element_type=jnp.float32) + blin_ref[...]


def _final(accp, g, dis, b4, batch8, Wlin, blin):
    return pl.pallas_call(
        _final_body,
        out_shape=jax.ShapeDtypeStruct((N_GRAPHS, N_CLASSES), jnp.float32),
    )(accp, g, dis, b4.reshape(1, D), batch8, Wlin,
      blin.reshape(1, N_CLASSES))


def kernel(x, edge_index, batch, W1, b1, W2, b2, W3, b3, W4, b4, Wlin, blin):
    src = edge_index[0].astype(jnp.int32)
    dst = edge_index[1].astype(jnp.int32)
    pad = PAD_EDGES - N_EDGES
    src_p = jnp.concatenate(
        [src, jnp.zeros((pad,), jnp.int32)]).reshape(NW, CH_PER_TILE, CHUNK)
    dst_p = jnp.concatenate(
        [dst, jnp.full((pad,), TRASH_ROW, jnp.int32)]
    ).reshape(NW, CH_PER_TILE, CHUNK)
    batch8 = jnp.broadcast_to(batch.astype(jnp.int32)[None, :], (8, N_NODES))

    degp = _deg_kernel(dst_p)
    g1, dis = _layer1(degp, x, W1)
    acc1 = _agg_kernel(g1, src_p, dst_p)
    g2 = _mid_layer(acc1, g1, dis, b1, W2)
    acc2 = _agg_kernel(g2, src_p, dst_p)
    g3 = _mid_layer(acc2, g2, dis, b2, W3)
    acc3 = _agg_kernel(g3, src_p, dst_p)
    g4 = _mid_layer(acc3, g3, dis, b3, W4)
    acc4 = _agg_kernel(g4, src_p, dst_p)
    return _final(acc4, g4, dis, b4, batch8, Wlin, blin)


# speedup vs baseline: 2.6775x; 2.6775x over previous
import functools

import jax
import jax.numpy as jnp
from jax import lax
from jax.experimental import pallas as pl
from jax.experimental.pallas import tpu as pltpu
from jax.experimental.pallas import tpu_sc as plsc

N_NODES = 10000
N_EDGES = 320000
D = 128
N_GRAPHS = 64
N_CLASSES = 10

NC = 2
NS = 16
NW = NC * NS

CHUNK = 125
CH_PER_TILE = 80
EDGES_PER_TILE = CH_PER_TILE * CHUNK
ACC_ROWS = 10112
ROWS_PER_TILE = ACC_ROWS // NS
ZCH = 104
WB_ROWS = 624

DEG_W = 16

_mesh = plsc.VectorSubcoreMesh(core_axis_name="c", subcore_axis_name="s")


def _fill_vmem(ref, rows, width, value):
    vec = jnp.full((16,), value, jnp.float32)

    def body(i, _):
        for j in range(width // 16):
            ref[i, pl.ds(j * 16, 16)] = vec
        return 0

    lax.fori_loop(0, rows, body, 0)


def _zero_spmem_slice(tile_id, zbuf, table):
    n_full = ROWS_PER_TILE // ZCH
    rem = ROWS_PER_TILE - n_full * ZCH
    base = tile_id * ROWS_PER_TILE
    for k in range(n_full):
        pltpu.sync_copy(zbuf.at[pl.ds(0, ZCH)],
                        table.at[pl.ds(base + k * ZCH, ZCH)])
    pltpu.sync_copy(zbuf.at[pl.ds(0, rem)],
                    table.at[pl.ds(base + n_full * ZCH, rem)])


def _writeback(c, s, table, out_hbm):
    base = s * WB_ROWS
    pltpu.sync_copy(table.at[pl.ds(base, WB_ROWS)],
                    out_hbm.at[c, pl.ds(base, WB_ROWS)])

    @pl.when(s == NS - 1)
    def _():
        tail = NS * WB_ROWS
        pltpu.sync_copy(table.at[pl.ds(tail, N_NODES - tail)],
                        out_hbm.at[c, pl.ds(tail, N_NODES - tail)])


@functools.partial(
    pl.kernel,
    out_type=jax.ShapeDtypeStruct((NC, N_NODES, DEG_W), jnp.float32),
    mesh=_mesh,
    scratch_types=[
        pltpu.VMEM((CH_PER_TILE, CHUNK), jnp.int32),
        pltpu.VMEM((CHUNK, DEG_W), jnp.float32),
        pltpu.VMEM_SHARED((ACC_ROWS, DEG_W), jnp.float32),
    ],
)
def _deg_kernel(dst_hbm, out_hbm, dstv, ones_v, table):
    c = lax.axis_index("c")
    s = lax.axis_index("s")
    wid = c * NS + s

    pltpu.sync_copy(dst_hbm.at[wid], dstv)

    _fill_vmem(ones_v, CHUNK, DEG_W, 0.0)
    _zero_spmem_slice(s, ones_v, table)
    _fill_vmem(ones_v, CHUNK, DEG_W, 1.0)

    plsc.subcore_barrier()

    def body(j, _):
        pltpu.sync_copy(ones_v, table.at[dstv.at[j]], add=True)
        return 0

    lax.fori_loop(0, CH_PER_TILE, body, 0)

    plsc.subcore_barrier()

    _writeback(c, s, table, out_hbm)


@functools.partial(
    pl.kernel,
    out_type=jax.ShapeDtypeStruct((NC, N_NODES, D), jnp.float32),
    mesh=_mesh,
    scratch_types=[
        pltpu.VMEM((CH_PER_TILE, CHUNK), jnp.int32),
        pltpu.VMEM((CH_PER_TILE, CHUNK), jnp.int32),
        pltpu.VMEM((CHUNK, D), jnp.float32),
        pltpu.VMEM_SHARED((ACC_ROWS, D), jnp.float32),
        pltpu.SemaphoreType.DMA,
    ],
)
def _agg_kernel(g_hbm, src_hbm, dst_hbm, out_hbm, srcv, dstv, bufa,
                acc, sema):
    c = lax.axis_index("c")
    s = lax.axis_index("s")
    wid = c * NS + s

    pltpu.sync_copy(src_hbm.at[wid], srcv)
    pltpu.sync_copy(dst_hbm.at[wid], dstv)

    _fill_vmem(bufa, CHUNK, D, 0.0)
    _zero_spmem_slice(s, bufa, acc)

    plsc.subcore_barrier()

    def body(j, _):
        pltpu.async_copy(g_hbm.at[srcv.at[j]], bufa, sema).wait()
        pltpu.sync_copy(bufa, acc.at[dstv.at[j]], add=True)
        return 0

    lax.fori_loop(0, CH_PER_TILE, body, 0)

    plsc.subcore_barrier()

    _writeback(c, s, acc, out_hbm)


ROW_BLK = 1000


def _l1_body(degp_ref, x_ref, w_ref, g_ref, dis_ref):
    deg = degp_ref[0, :, 0:1] + degp_ref[1, :, 0:1] + 1.0
    dis = lax.rsqrt(deg)
    h = jnp.dot(x_ref[...], w_ref[...], preferred_element_type=jnp.float32)
    g_ref[...] = dis * h
    dis_ref[...] = dis


def _layer1(degp, x, W1):
    return pl.pallas_call(
        _l1_body,
        grid=(N_NODES // ROW_BLK,),
        in_specs=[
            pl.BlockSpec((NC, ROW_BLK, DEG_W), lambda i: (0, i, 0)),
            pl.BlockSpec((ROW_BLK, D), lambda i: (i, 0)),
            pl.BlockSpec((D, D), lambda i: (0, 0)),
        ],
        out_specs=[
            pl.BlockSpec((ROW_BLK, D), lambda i: (i, 0)),
            pl.BlockSpec((ROW_BLK, 1), lambda i: (i, 0)),
        ],
        out_shape=[
            jax.ShapeDtypeStruct((N_NODES, D), jnp.float32),
            jax.ShapeDtypeStruct((N_NODES, 1), jnp.float32),
        ],
    )(degp, x, W1)


def _mid_body(accp_ref, g_ref, dis_ref, b_ref, w_ref, out_ref):
    acc = accp_ref[0] + accp_ref[1]
    dis = dis_ref[...]
    a = jnp.maximum(dis * (acc + g_ref[...]) + b_ref[...], 0.0)
    out_ref[...] = dis * jnp.dot(a, w_ref[...],
                                 preferred_element_type=jnp.float32)


def _mid_layer(accp, g, dis, b, W):
    return pl.pallas_call(
        _mid_body,
        grid=(N_NODES // ROW_BLK,),
        in_specs=[
            pl.BlockSpec((NC, ROW_BLK, D), lambda i: (0, i, 0)),
            pl.BlockSpec((ROW_BLK, D), lambda i: (i, 0)),
            pl.BlockSpec((ROW_BLK, 1), lambda i: (i, 0)),
            pl.BlockSpec((1, D), lambda i: (0, 0)),
            pl.BlockSpec((D, D), lambda i: (0, 0)),
        ],
        out_specs=pl.BlockSpec((ROW_BLK, D), lambda i: (i, 0)),
        out_shape=jax.ShapeDtypeStruct((N_NODES, D), jnp.float32),
    )(accp, g, dis, b.reshape(1, D), W)


def _final_body(accp_ref, g_ref, dis_ref, b_ref, batch_ref, wlin_ref,
                blin_ref, out_ref):
    acc = accp_ref[0] + accp_ref[1]
    dis = dis_ref[...]
    a = jnp.maximum(dis * (acc + g_ref[...]) + b_ref[...], 0.0)
    batch_row = batch_ref[0:1, :]
    gids = lax.broadcasted_iota(jnp.int32, (N_GRAPHS, N_NODES), 0)
    mask = (gids == batch_row).astype(jnp.float32)
    sums = jnp.dot(mask, a, preferred_element_type=jnp.float32)
    cnts = jnp.sum(mask, axis=1, keepdims=True)
    pooled = sums / jnp.maximum(cnts, 1.0)
    out_ref[...] = jnp.dot(pooled, wlin_ref[...],
                           preferred_element_type=jnp.float32) + blin_ref[...]


def _final(accp, g, dis, b4, batch8, Wlin, blin):
    return pl.pallas_call(
        _final_body,
        out_shape=jax.ShapeDtypeStruct((N_GRAPHS, N_CLASSES), jnp.float32),
    )(accp, g, dis, b4.reshape(1, D), batch8, Wlin,
      blin.reshape(1, N_CLASSES))


def kernel(x, edge_index, batch, W1, b1, W2, b2, W3, b3, W4, b4, Wlin, blin):
    src_p = edge_index[0].astype(jnp.int32).reshape(NW, CH_PER_TILE, CHUNK)
    dst_p = edge_index[1].astype(jnp.int32).reshape(NW, CH_PER_TILE, CHUNK)
    batch8 = jnp.broadcast_to(batch.astype(jnp.int32)[None, :], (8, N_NODES))

    degp = _deg_kernel(dst_p)
    g1, dis = _layer1(degp, x, W1)
    acc1 = _agg_kernel(g1, src_p, dst_p)
    g2 = _mid_layer(acc1, g1, dis, b1, W2)
    acc2 = _agg_kernel(g2, src_p, dst_p)
    g3 = _mid_layer(acc2, g2, dis, b2, W3)
    acc3 = _agg_kernel(g3, src_p, dst_p)
    g4 = _mid_layer(acc3, g3, dis, b3, W4)
    acc4 = _agg_kernel(g4, src_p, dst_p)
    return _final(acc4, g4, dis, b4, batch8, Wlin, blin)
